# baseline (speedup 1.0000x reference)
import jax
import jax.numpy as jnp
from jax import lax
from jax.experimental import pallas as pl
from jax.experimental.pallas import tpu as pltpu
from jax.experimental.pallas import tpu_sc as plsc

N = 10000
E = 320000
NC = 2
NS = 16
CH = 128
NCHUNK = E // CH
NPAD = 10240
RPT = NPAD // NS
FH = 128 // NC
BLK = 1000
GRID = N // BLK

_sc_mesh = plsc.VectorSubcoreMesh(
    core_axis_name="c", subcore_axis_name="s", num_cores=NC, num_subcores=NS)
_sc_params = pltpu.CompilerParams(needs_layout_passes=False,
                                 use_tc_tiling_on_sc=False)


def _deg_body(dst3, dis_out, idx_d, partial, part16, disbuf, part_sh):
    cid = lax.axis_index("c")
    sid = lax.axis_index("s")
    zero16 = jnp.zeros((16,), jnp.float32)
    one16 = jnp.full((16,), 1.0, jnp.float32)
    r0 = sid * RPT

    def zp(i, _):
        partial[pl.ds(i * 16, 16)] = zero16
        return 0
    lax.fori_loop(0, NPAD // 16, zp, 0)

    def dbody(i, _):
        chunk = sid + i * NS

        @pl.when(chunk < NCHUNK)
        def _():
            pltpu.sync_copy(dst3.at[chunk, 0], idx_d)
            for k in range(CH // 16):
                iv = idx_d[pl.ds(k * 16, 16)]
                plsc.addupdate_scatter(partial, [iv], one16)
        return 0
    lax.fori_loop(0, (NCHUNK + NS - 1) // NS, dbody, 0)
    pltpu.sync_copy(partial, part_sh.at[sid])
    plsc.subcore_barrier()

    pltpu.sync_copy(part_sh.at[:, pl.ds(r0, RPT)], part16)

    def rbody(j, _):
        d = jnp.full((16,), 1.0, jnp.float32)
        for r in range(NS):
            d = d + part16[r, pl.ds(j * 16, 16)]
        iv = plsc.bitcast(d, jnp.int32)
        yi = jnp.int32(0x5F3759DF) - lax.shift_right_arithmetic(iv, 1)
        ds_v = plsc.bitcast(yi, jnp.float32)
        for _ in range(3):
            ds_v = ds_v * (1.5 - 0.5 * d * ds_v * ds_v)
        disbuf[pl.ds(j * 16, 16)] = ds_v
        return 0
    lax.fori_loop(0, RPT // 16, rbody, 0)

    @pl.when(cid == 0)
    def _():
        pltpu.sync_copy(disbuf, dis_out.at[pl.ds(r0, RPT)])


_deg_kernel = pl.kernel(
    _deg_body,
    out_type=jax.ShapeDtypeStruct((NPAD,), jnp.float32),
    mesh=_sc_mesh,
    compiler_params=_sc_params,
    scratch_types=[
        pltpu.VMEM((CH,), jnp.int32),
        pltpu.VMEM((NPAD,), jnp.float32),
        pltpu.VMEM((NS, RPT), jnp.float32),
        pltpu.VMEM((RPT,), jnp.float32),
        pltpu.VMEM_SHARED((NS, NPAD), jnp.float32),
    ],
)


def _agg_body(src3, dst3, y_hbm, s_out, idx_s, idx_d, rows,
              table_sh, acc_sh):
    cid = lax.axis_index("c")
    sid = lax.axis_index("s")
    r0 = sid * RPT

    SB = 128
    for t in range(RPT // SB):
        sl = pl.ds(r0 + t * SB, SB)
        pltpu.sync_copy(y_hbm.at[cid, sl], rows.at[pl.ds(0, SB)])
        pltpu.sync_copy(rows.at[pl.ds(0, SB)], table_sh.at[sl])
        pltpu.sync_copy(rows.at[pl.ds(0, SB)], acc_sh.at[sl])
    plsc.subcore_barrier()

    def ebody(i, _):
        chunk = sid + i * NS

        @pl.when(chunk < NCHUNK)
        def _():
            pltpu.sync_copy(src3.at[chunk, 0], idx_s)
            pltpu.sync_copy(dst3.at[chunk, 0], idx_d)
            pltpu.sync_copy(table_sh.at[idx_s], rows)
            pltpu.sync_copy(rows, acc_sh.at[idx_d], add=True)
        return 0
    lax.fori_loop(0, (NCHUNK + NS - 1) // NS, ebody, 0)
    plsc.subcore_barrier()

    SB2 = 128
    for t in range(RPT // SB2):
        sl = pl.ds(r0 + t * SB2, SB2)
        pltpu.sync_copy(acc_sh.at[sl], rows.at[pl.ds(0, SB2)])
        pltpu.sync_copy(rows.at[pl.ds(0, SB2)], s_out.at[cid, sl])


_agg_kernel = pl.kernel(
    _agg_body,
    out_type=jax.ShapeDtypeStruct((NC, NPAD, FH), jnp.float32),
    mesh=_sc_mesh,
    compiler_params=_sc_params,
    scratch_types=[
        pltpu.VMEM((CH,), jnp.int32),
        pltpu.VMEM((CH,), jnp.int32),
        pltpu.VMEM((CH, FH), jnp.float32),
        pltpu.VMEM_SHARED((NPAD, FH), jnp.float32),
        pltpu.VMEM_SHARED((NPAD, FH), jnp.float32),
    ],
)


def _prep_body(x_ref, wg_ref, bg_ref, gw_ref, dis_ref, y_out):
    g = jnp.dot(x_ref[...], wg_ref[...],
                preferred_element_type=jnp.float32) + bg_ref[...]
    gi = jax.nn.sigmoid(g[:, :128])
    gc = jnp.tanh(g[:, 128:256])
    go = jax.nn.sigmoid(g[:, 256:])
    hs = go * jnp.tanh(gi * gc)
    xw = jnp.dot(hs, gw_ref[...], preferred_element_type=jnp.float32)
    y = xw * dis_ref[...]
    y_out[0] = y[:, :FH]
    y_out[1] = y[:, FH:]


_prep = pl.pallas_call(
    _prep_body,
    grid=(GRID,),
    in_specs=[
        pl.BlockSpec((BLK, 128), lambda i: (i, 0)),
        pl.BlockSpec((128, 384), lambda i: (0, 0)),
        pl.BlockSpec((1, 384), lambda i: (0, 0)),
        pl.BlockSpec((128, 128), lambda i: (0, 0)),
        pl.BlockSpec((BLK, 1), lambda i: (i, 0)),
    ],
    out_specs=pl.BlockSpec((NC, BLK, FH), lambda i: (0, i, 0)),
    out_shape=jax.ShapeDtypeStruct((NC, NPAD, FH), jnp.float32),
)


def _post_body(s_ref, dis_ref, gb_ref, bnsc_ref, bnsh_ref,
               lw_ref, lb_ref, out_ref):
    dis = dis_ref[...]
    acc = jnp.zeros((BLK, 1), jnp.float32)
    for c in range(NC):
        sl = slice(c * FH, (c + 1) * FH)
        aggc = dis * s_ref[c] + gb_ref[...][:, sl]
        h = jnp.maximum(aggc, 0.0)
        h = h * bnsc_ref[...][:, sl] + bnsh_ref[...][:, sl]
        acc = acc + jnp.sum(h * lw_ref[...][:, sl], axis=1, keepdims=True)
    out_ref[...] = acc + lb_ref[...]


_post = pl.pallas_call(
    _post_body,
    grid=(GRID,),
    in_specs=[
        pl.BlockSpec((NC, BLK, FH), lambda i: (0, i, 0)),
        pl.BlockSpec((BLK, 1), lambda i: (i, 0)),
        pl.BlockSpec((1, 128), lambda i: (0, 0)),
        pl.BlockSpec((1, 128), lambda i: (0, 0)),
        pl.BlockSpec((1, 128), lambda i: (0, 0)),
        pl.BlockSpec((1, 128), lambda i: (0, 0)),
        pl.BlockSpec((1, 1), lambda i: (0, 0)),
    ],
    out_specs=pl.BlockSpec((BLK, 1), lambda i: (i, 0)),
    out_shape=jax.ShapeDtypeStruct((N, 1), jnp.float32),
)


def kernel(x, edge_index, edge_weight,
           W_i, b_i, cw_i, cb_i,
           W_f, b_f, cw_f, cb_f,
           W_c, b_c, cw_c, cb_c,
           W_o, b_o, cw_o, cb_o,
           gcn_W, gcn_b,
           bn_gamma, bn_beta, bn_mean, bn_var,
           lin_W, lin_b):
    src3 = edge_index[0].reshape(NCHUNK, 1, CH)
    dst3 = edge_index[1].reshape(NCHUNK, 1, CH)
    Wg = jnp.concatenate([W_i, W_c, W_o], axis=1)
    bg = jnp.concatenate([cb_i[None, :] + b_i, cb_c[None, :] + b_c,
                          cb_o[None, :] + b_o], axis=1)
    bn_sc = (bn_gamma / jnp.sqrt(bn_var + 1e-5)).reshape(1, 128)
    bn_sh = (bn_beta - bn_mean * bn_sc[0]).reshape(1, 128)
    gb = gcn_b.reshape(1, 128)
    lw = lin_W.reshape(1, 128)
    lb = lin_b.reshape(1, 1)

    dis = _deg_kernel(dst3).reshape(NPAD, 1)
    y = _prep(x, Wg, bg, gcn_W, dis)
    s = _agg_kernel(src3, dst3, y)
    return _post(s, dis, gb, bn_sc, bn_sh, lw, lb)


# speedup vs baseline: 1.1759x; 1.1759x over previous
import jax
import jax.numpy as jnp
from jax import lax
from jax.experimental import pallas as pl
from jax.experimental.pallas import tpu as pltpu
from jax.experimental.pallas import tpu_sc as plsc

N = 10000
E = 320000
NC = 2
NS = 16
CH = 128
NCHUNK = E // CH
NPAD = 10240
RPT = NPAD // NS
FH = 128 // NC
BLK = 1000
GRID = N // BLK

_sc_mesh = plsc.VectorSubcoreMesh(
    core_axis_name="c", subcore_axis_name="s", num_cores=NC, num_subcores=NS)
_sc_params = pltpu.CompilerParams(needs_layout_passes=False,
                                 use_tc_tiling_on_sc=False)


EPT = E // NS


def _deg_body(dstf, dis_out, bigidx, partial, part16, disbuf, part_sh):
    cid = lax.axis_index("c")
    sid = lax.axis_index("s")
    zero16 = jnp.zeros((16,), jnp.float32)
    one16 = jnp.full((16,), 1.0, jnp.float32)
    r0 = sid * RPT

    def zp(i, _):
        partial[pl.ds(i * 16, 16)] = zero16
        return 0
    lax.fori_loop(0, NPAD // 16, zp, 0)

    pltpu.sync_copy(dstf.at[pl.ds(sid * EPT, EPT)], bigidx)

    def dbody(g, _):
        iv = bigidx[pl.ds(g * 16, 16)]
        plsc.addupdate_scatter(partial, [iv], one16)
        return 0
    lax.fori_loop(0, EPT // 16, dbody, 0)
    pltpu.sync_copy(partial, part_sh.at[sid])
    plsc.subcore_barrier()

    pltpu.sync_copy(part_sh.at[:, pl.ds(r0, RPT)], part16)

    def rbody(j, _):
        d = jnp.full((16,), 1.0, jnp.float32)
        for r in range(NS):
            d = d + part16[r, pl.ds(j * 16, 16)]
        iv = plsc.bitcast(d, jnp.int32)
        yi = jnp.int32(0x5F3759DF) - lax.shift_right_arithmetic(iv, 1)
        ds_v = plsc.bitcast(yi, jnp.float32)
        for _ in range(3):
            ds_v = ds_v * (1.5 - 0.5 * d * ds_v * ds_v)
        disbuf[pl.ds(j * 16, 16)] = ds_v
        return 0
    lax.fori_loop(0, RPT // 16, rbody, 0)

    @pl.when(cid == 0)
    def _():
        pltpu.sync_copy(disbuf, dis_out.at[pl.ds(r0, RPT)])


_deg_kernel = pl.kernel(
    _deg_body,
    out_type=jax.ShapeDtypeStruct((NPAD,), jnp.float32),
    mesh=_sc_mesh,
    compiler_params=_sc_params,
    scratch_types=[
        pltpu.VMEM((EPT,), jnp.int32),
        pltpu.VMEM((NPAD,), jnp.float32),
        pltpu.VMEM((NS, RPT), jnp.float32),
        pltpu.VMEM((RPT,), jnp.float32),
        pltpu.VMEM_SHARED((NS, NPAD), jnp.float32),
    ],
)


def _agg_body(src3, dst3, xw_hbm, dis_hbm, s_out, idx_s, idx_d, rows, disbuf,
              table_sh, acc_sh):
    cid = lax.axis_index("c")
    sid = lax.axis_index("s")
    r0 = sid * RPT

    pltpu.sync_copy(dis_hbm.at[pl.ds(r0, RPT)], disbuf)
    SB = 128
    for t in range(RPT // SB):
        sl = pl.ds(r0 + t * SB, SB)
        pltpu.sync_copy(xw_hbm.at[cid, sl], rows.at[pl.ds(0, SB)])

        def scale_grp(g, _):
            dv = disbuf[pl.ds(t * SB + g * 16, 16)]
            for k in range(16):
                dsc = jnp.full((16,), dv[k], jnp.float32)
                for j in range(FH // 16):
                    rows[g * 16 + k, pl.ds(j * 16, 16)] = (
                        rows[g * 16 + k, pl.ds(j * 16, 16)] * dsc)
            return 0
        lax.fori_loop(0, SB // 16, scale_grp, 0)
        pltpu.sync_copy(rows.at[pl.ds(0, SB)], table_sh.at[sl])
        pltpu.sync_copy(rows.at[pl.ds(0, SB)], acc_sh.at[sl])
    plsc.subcore_barrier()

    def ebody(i, _):
        chunk = sid + i * NS

        @pl.when(chunk < NCHUNK)
        def _():
            pltpu.sync_copy(src3.at[chunk, 0], idx_s)
            pltpu.sync_copy(dst3.at[chunk, 0], idx_d)
            pltpu.sync_copy(table_sh.at[idx_s], rows)
            pltpu.sync_copy(rows, acc_sh.at[idx_d], add=True)
        return 0
    lax.fori_loop(0, (NCHUNK + NS - 1) // NS, ebody, 0)
    plsc.subcore_barrier()

    SB2 = 128
    for t in range(RPT // SB2):
        sl = pl.ds(r0 + t * SB2, SB2)
        pltpu.sync_copy(acc_sh.at[sl], rows.at[pl.ds(0, SB2)])
        pltpu.sync_copy(rows.at[pl.ds(0, SB2)], s_out.at[cid, sl])


_agg_kernel = pl.kernel(
    _agg_body,
    out_type=jax.ShapeDtypeStruct((NC, NPAD, FH), jnp.float32),
    mesh=_sc_mesh,
    compiler_params=_sc_params,
    scratch_types=[
        pltpu.VMEM((CH,), jnp.int32),
        pltpu.VMEM((CH,), jnp.int32),
        pltpu.VMEM((CH, FH), jnp.float32),
        pltpu.VMEM((RPT,), jnp.float32),
        pltpu.VMEM_SHARED((NPAD, FH), jnp.float32),
        pltpu.VMEM_SHARED((NPAD, FH), jnp.float32),
    ],
)


def _prep_body(x_ref, wg_ref, bg_ref, gw_ref, xw_out):
    g = jnp.dot(x_ref[...], wg_ref[...],
                preferred_element_type=jnp.float32) + bg_ref[...]
    gi = jax.nn.sigmoid(g[:, :128])
    gc = jnp.tanh(g[:, 128:256])
    go = jax.nn.sigmoid(g[:, 256:])
    hs = go * jnp.tanh(gi * gc)
    xw = jnp.dot(hs, gw_ref[...], preferred_element_type=jnp.float32)
    xw_out[0] = xw[:, :FH]
    xw_out[1] = xw[:, FH:]


_prep = pl.pallas_call(
    _prep_body,
    grid=(GRID,),
    in_specs=[
        pl.BlockSpec((BLK, 128), lambda i: (i, 0)),
        pl.BlockSpec((128, 384), lambda i: (0, 0)),
        pl.BlockSpec((1, 384), lambda i: (0, 0)),
        pl.BlockSpec((128, 128), lambda i: (0, 0)),
    ],
    out_specs=pl.BlockSpec((NC, BLK, FH), lambda i: (0, i, 0)),
    out_shape=jax.ShapeDtypeStruct((NC, NPAD, FH), jnp.float32),
)


def _post_body(s_ref, dis_ref, gb_ref, bnsc_ref, bnsh_ref,
               lw_ref, lb_ref, out_ref):
    dis = dis_ref[...]
    acc = jnp.zeros((BLK, 1), jnp.float32)
    for c in range(NC):
        sl = slice(c * FH, (c + 1) * FH)
        aggc = dis * s_ref[c] + gb_ref[...][:, sl]
        h = jnp.maximum(aggc, 0.0)
        h = h * bnsc_ref[...][:, sl] + bnsh_ref[...][:, sl]
        acc = acc + jnp.sum(h * lw_ref[...][:, sl], axis=1, keepdims=True)
    out_ref[...] = acc + lb_ref[...]


_post = pl.pallas_call(
    _post_body,
    grid=(GRID,),
    in_specs=[
        pl.BlockSpec((NC, BLK, FH), lambda i: (0, i, 0)),
        pl.BlockSpec((BLK, 1), lambda i: (i, 0)),
        pl.BlockSpec((1, 128), lambda i: (0, 0)),
        pl.BlockSpec((1, 128), lambda i: (0, 0)),
        pl.BlockSpec((1, 128), lambda i: (0, 0)),
        pl.BlockSpec((1, 128), lambda i: (0, 0)),
        pl.BlockSpec((1, 1), lambda i: (0, 0)),
    ],
    out_specs=pl.BlockSpec((BLK, 1), lambda i: (i, 0)),
    out_shape=jax.ShapeDtypeStruct((N, 1), jnp.float32),
)


def kernel(x, edge_index, edge_weight,
           W_i, b_i, cw_i, cb_i,
           W_f, b_f, cw_f, cb_f,
           W_c, b_c, cw_c, cb_c,
           W_o, b_o, cw_o, cb_o,
           gcn_W, gcn_b,
           bn_gamma, bn_beta, bn_mean, bn_var,
           lin_W, lin_b):
    src3 = edge_index[0].reshape(NCHUNK, 1, CH)
    dst3 = edge_index[1].reshape(NCHUNK, 1, CH)
    Wg = jnp.concatenate([W_i, W_c, W_o], axis=1)
    bg = jnp.concatenate([cb_i[None, :] + b_i, cb_c[None, :] + b_c,
                          cb_o[None, :] + b_o], axis=1)
    bn_sc = (bn_gamma / jnp.sqrt(bn_var + 1e-5)).reshape(1, 128)
    bn_sh = (bn_beta - bn_mean * bn_sc[0]).reshape(1, 128)
    gb = gcn_b.reshape(1, 128)
    lw = lin_W.reshape(1, 128)
    lb = lin_b.reshape(1, 1)

    dis = _deg_kernel(edge_index[1])
    xw = _prep(x, Wg, bg, gcn_W)
    s = _agg_kernel(src3, dst3, xw, dis)
    return _post(s, dis.reshape(NPAD, 1), gb, bn_sc, bn_sh, lw, lb)
